# baseline (speedup 1.0000x reference)
import functools

import jax
import jax.numpy as jnp
from jax import lax
from jax.experimental import pallas as pl
from jax.experimental.pallas import tpu as pltpu
from jax.experimental.pallas import tpu_sc as plsc

N = 100000
E = 100000
F = 128

NPAD = 102400
EPAD = 104448
NG = 8
GR = NPAD // NG
ACC_R = GR + 16
WBR = GR // 16
EPS = EPAD // 16
GWB = 48
PAIRS = EPS // (2 * GWB)
EPW = EPAD // 32
GW = 96
NWIN = EPW // GW
VREGS = EPW // 16
B_ROWS = 1024
NB = NPAD // B_ROWS

_mesh = plsc.VectorSubcoreMesh(core_axis_name="c", subcore_axis_name="s",
                               num_cores=2)


def _mm_body(x_ref, w_ref, y_ref):
    y_ref[...] = jnp.dot(x_ref[...], w_ref[...],
                         preferred_element_type=jnp.float32)


def _matmul(x, w):
    grid = (pl.cdiv(N, B_ROWS),)
    return pl.pallas_call(
        _mm_body,
        grid=grid,
        in_specs=[pl.BlockSpec((B_ROWS, F), lambda i: (i, 0)),
                  pl.BlockSpec((F, F), lambda i: (0, 0))],
        out_specs=pl.BlockSpec((B_ROWS, F), lambda i: (i, 0)),
        out_shape=jax.ShapeDtypeStruct((N, F), jnp.float32),
    )(x, w)


def _scatter_body(y_h, row_h, col_h, zer_h, out_h,
                  row_v, col_v, buf_a, buf_b, acc,
                  sem_a, sem_b, sem_sa, sem_sb):
    c = lax.axis_index("c")
    s = lax.axis_index("s")
    base = s * EPS
    pltpu.sync_copy(row_h.at[pl.ds(base, EPS)], row_v)
    pltpu.sync_copy(col_h.at[pl.ds(base, EPS)], col_v)
    zer16 = jnp.full((16,), 0, jnp.int32)
    lane = lax.iota(jnp.int32, 16)
    dummy_v = jnp.full((16,), GR, jnp.int32) + lane

    for p in range(NG // 2):
        g = c * (NG // 2) + p
        g_lo = g * GR
        lo_v = zer16 + g_lo
        hi_v = lo_v + GR

        pltpu.sync_copy(zer_h, acc.at[pl.ds(s * WBR, WBR)])

        @pl.when(s == 0)
        def _zero_dummy():
            pltpu.sync_copy(zer_h.at[pl.ds(0, 16)],
                            acc.at[pl.ds(GR, 16)])

        plsc.subcore_barrier()

        def do_adds(off, buf, sem_s):
            for jj in range(GWB // 16):
                r = row_v[pl.ds(off + jj * 16, 16)]
                m = (r >= lo_v) & (r < hi_v)
                d = jnp.where(m, r - lo_v, dummy_v)
                pltpu.async_copy(buf.at[pl.ds(jj * 16, 16)],
                                 acc.at[d], sem_s, add=True)

        def drain(buf, sem_s):
            pltpu.make_async_copy(buf, acc.at[pl.ds(0, GWB)], sem_s).wait()

        def gs_body(q, _):
            off0 = q * (2 * GWB)
            off1 = off0 + GWB
            c0 = pltpu.async_copy(
                y_h.at[col_v.at[pl.ds(off0, GWB)]], buf_a, sem_a)
            c1 = pltpu.async_copy(
                y_h.at[col_v.at[pl.ds(off1, GWB)]], buf_b, sem_b)
            c0.wait()
            do_adds(off0, buf_a, sem_sa)
            c1.wait()
            do_adds(off1, buf_b, sem_sb)
            drain(buf_a, sem_sa)
            drain(buf_b, sem_sb)
            return 0

        lax.fori_loop(0, PAIRS, gs_body, 0)
        plsc.subcore_barrier()
        pltpu.sync_copy(acc.at[pl.ds(s * WBR, WBR)],
                        out_h.at[pl.ds(g_lo + s * WBR, WBR)])


def _scatter_add(y, rowp, colp, zer):
    f = functools.partial(
        pl.kernel,
        mesh=_mesh,
        out_type=jax.ShapeDtypeStruct((NPAD, F), jnp.float32),
        scratch_types=[
            pltpu.VMEM((EPS,), jnp.int32),
            pltpu.VMEM((EPS,), jnp.int32),
            pltpu.VMEM((GWB, F), jnp.float32),
            pltpu.VMEM((GWB, F), jnp.float32),
            pltpu.VMEM_SHARED((ACC_R, F), jnp.float32),
            pltpu.SemaphoreType.DMA,
            pltpu.SemaphoreType.DMA,
            pltpu.SemaphoreType.DMA,
            pltpu.SemaphoreType.DMA,
        ],
    )(_scatter_body)
    return f(y, rowp, colp, zer)


def _scores_body(o_ref, b_ref, a1_ref, a2_ref, s1_ref, s2_ref):
    h = jax.nn.relu(o_ref[...] + b_ref[...])
    s1_ref[...] = jnp.sum(h * a1_ref[...], axis=1).reshape(8, F)
    s2_ref[...] = jnp.sum(h * a2_ref[...], axis=1).reshape(8, F)


def _scores(out0, bias2d, a1, a2):
    grid = (NB,)
    return pl.pallas_call(
        _scores_body,
        grid=grid,
        in_specs=[pl.BlockSpec((B_ROWS, F), lambda i: (i, 0)),
                  pl.BlockSpec((1, F), lambda i: (0, 0)),
                  pl.BlockSpec((1, F), lambda i: (0, 0)),
                  pl.BlockSpec((1, F), lambda i: (0, 0))],
        out_specs=[pl.BlockSpec((8, F), lambda i: (i, 0)),
                   pl.BlockSpec((8, F), lambda i: (i, 0))],
        out_shape=[jax.ShapeDtypeStruct((NPAD // F, F), jnp.float32),
                   jax.ShapeDtypeStruct((NPAD // F, F), jnp.float32)],
    )(out0, bias2d, a1, a2)


def _edge_body(row_h, col_h, s1_h, s2_h, e_h, row_v, col_v, a_v, b_v, sem):
    c = lax.axis_index("c")
    s = lax.axis_index("s")
    gw = c * 16 + s
    base = gw * EPW
    pltpu.sync_copy(row_h.at[pl.ds(base, EPW)], row_v)
    pltpu.sync_copy(col_h.at[pl.ds(base, EPW)], col_v)

    def gbody(w, _):
        off = w * GW
        c1 = pltpu.async_copy(s1_h.at[row_v.at[pl.ds(off, GW)]],
                              a_v.at[pl.ds(off, GW)], sem)
        c2 = pltpu.async_copy(s2_h.at[col_v.at[pl.ds(off, GW)]],
                              b_v.at[pl.ds(off, GW)], sem)
        c1.wait()
        c2.wait()
        return 0

    lax.fori_loop(0, NWIN, gbody, 0)

    def cbody(j, _):
        a = a_v[pl.ds(j * 16, 16)]
        b = b_v[pl.ds(j * 16, 16)]
        e = a + b
        a_v[pl.ds(j * 16, 16)] = jnp.where(e >= 0.0, e, e * 0.2)
        return 0

    lax.fori_loop(0, VREGS, cbody, 0)
    pltpu.sync_copy(a_v, e_h.at[pl.ds(base, EPW)])


def _edge_scores(rowp, colp, s1, s2):
    f = functools.partial(
        pl.kernel,
        mesh=_mesh,
        out_type=jax.ShapeDtypeStruct((EPAD,), jnp.float32),
        scratch_types=[
            pltpu.VMEM((EPW,), jnp.int32),
            pltpu.VMEM((EPW,), jnp.int32),
            pltpu.VMEM((EPW,), jnp.float32),
            pltpu.VMEM((EPW,), jnp.float32),
            pltpu.SemaphoreType.DMA,
        ],
    )(_edge_body)
    return f(rowp, colp, s1, s2)


def _softmax_body(e_ref, al_ref):
    r = lax.broadcasted_iota(jnp.int32, (EPAD // F, F), 0)
    l = lax.broadcasted_iota(jnp.int32, (EPAD // F, F), 1)
    valid = (r * F + l) < E
    em = jnp.where(valid, e_ref[...], -1e30)
    m = jnp.max(em)
    p = jnp.where(valid, jnp.exp(em - m), 0.0)
    al_ref[...] = p * (1.0 / jnp.sum(p))


def _softmax(e2d):
    return pl.pallas_call(
        _softmax_body,
        out_shape=jax.ShapeDtypeStruct((EPAD // F, F), jnp.float32),
    )(e2d)


def _scale_body(o_ref, b_ref, al_ref, out_ref):
    h = jax.nn.relu(o_ref[...] + b_ref[...])
    out_ref[...] = h * al_ref[...]


def _scale(out0, bias2d, alpha_col):
    grid = (pl.cdiv(N, B_ROWS),)
    return pl.pallas_call(
        _scale_body,
        grid=grid,
        in_specs=[pl.BlockSpec((B_ROWS, F), lambda i: (i, 0)),
                  pl.BlockSpec((1, F), lambda i: (0, 0)),
                  pl.BlockSpec((B_ROWS, 1), lambda i: (i, 0))],
        out_specs=pl.BlockSpec((B_ROWS, F), lambda i: (i, 0)),
        out_shape=jax.ShapeDtypeStruct((N, F), jnp.float32),
    )(out0, bias2d, alpha_col)


def kernel(x, edge_index, weight, att, bias):
    row32 = edge_index[0].astype(jnp.int32)
    col32 = edge_index[1].astype(jnp.int32)
    pad = EPAD - E
    pad_i = jnp.arange(pad, dtype=jnp.int32)
    rowp = jnp.concatenate([row32, N + pad_i % (NPAD - N)])
    colp = jnp.concatenate([col32, pad_i % N])
    zer = jnp.zeros((WBR, F), jnp.float32)
    bias2d = bias.reshape(1, F)
    a1 = att[:, :F]
    a2 = att[:, F:]

    y = _matmul(x, weight)
    out0 = _scatter_add(y, rowp, colp, zer)
    s1, s2 = _scores(out0, bias2d, a1, a2)
    e = _edge_scores(rowp, colp, s1.reshape(NPAD), s2.reshape(NPAD))
    alpha2d = _softmax(e.reshape(EPAD // F, F))
    out = _scale(out0, bias2d, alpha2d.reshape(EPAD, 1))
    return out


# speedup vs baseline: 1.0057x; 1.0057x over previous
import functools

import jax
import jax.numpy as jnp
from jax import lax
from jax.experimental import pallas as pl
from jax.experimental.pallas import tpu as pltpu
from jax.experimental.pallas import tpu_sc as plsc

N = 100000
E = 100000
F = 128

NPAD = 102400
EPAD = 102400
NG = 10
GR = NPAD // NG
ACC_R = GR + 16
WBR = GR // 16
EPS = EPAD // 16
GWB = 128
PAIRS = EPS // (2 * GWB)
EPW = EPAD // 32
GW = 128
NWIN = EPW // GW
VREGS = EPW // 16
B_ROWS = 1024
NB = NPAD // B_ROWS

_mesh = plsc.VectorSubcoreMesh(core_axis_name="c", subcore_axis_name="s",
                               num_cores=2)


def _mm_body(x_ref, w_ref, y_ref):
    y_ref[...] = jnp.dot(x_ref[...], w_ref[...],
                         preferred_element_type=jnp.float32)


def _matmul(x, w):
    grid = (pl.cdiv(N, B_ROWS),)
    return pl.pallas_call(
        _mm_body,
        grid=grid,
        in_specs=[pl.BlockSpec((B_ROWS, F), lambda i: (i, 0)),
                  pl.BlockSpec((F, F), lambda i: (0, 0))],
        out_specs=pl.BlockSpec((B_ROWS, F), lambda i: (i, 0)),
        out_shape=jax.ShapeDtypeStruct((N, F), jnp.float32),
    )(x, w)


def _scatter_body(y_h, row_h, col_h, zer_h, out_h,
                  row_v, col_v, buf_a, buf_b, dref_a, dref_b, acc,
                  sem_a, sem_b, sem_sa, sem_sb):
    c = lax.axis_index("c")
    s = lax.axis_index("s")
    base = s * EPS
    pltpu.sync_copy(row_h.at[pl.ds(base, EPS)], row_v)
    pltpu.sync_copy(col_h.at[pl.ds(base, EPS)], col_v)
    zer16 = jnp.full((16,), 0, jnp.int32)
    lane = lax.iota(jnp.int32, 16)
    dummy_v = jnp.full((16,), GR, jnp.int32) + lane

    for p in range(NG // 2):
        g = c * (NG // 2) + p
        g_lo = g * GR
        lo_v = zer16 + g_lo
        hi_v = lo_v + GR

        pltpu.sync_copy(zer_h, acc.at[pl.ds(s * WBR, WBR)])

        @pl.when(s == 0)
        def _zero_dummy():
            pltpu.sync_copy(zer_h.at[pl.ds(0, 16)],
                            acc.at[pl.ds(GR, 16)])

        plsc.subcore_barrier()

        def do_adds(off, buf, dref, sem_s):
            for jj in range(GWB // 16):
                r = row_v[pl.ds(off + jj * 16, 16)]
                m = (r >= lo_v) & (r < hi_v)
                dref[pl.ds(jj * 16, 16)] = jnp.where(m, r - lo_v, dummy_v)
            pltpu.async_copy(buf, acc.at[dref], sem_s, add=True)

        def drain(buf, sem_s):
            pltpu.make_async_copy(buf, acc.at[pl.ds(0, GWB)], sem_s).wait()

        def gs_body(q, _):
            off0 = q * (2 * GWB)
            off1 = off0 + GWB
            c0 = pltpu.async_copy(
                y_h.at[col_v.at[pl.ds(off0, GWB)]], buf_a, sem_a)
            c1 = pltpu.async_copy(
                y_h.at[col_v.at[pl.ds(off1, GWB)]], buf_b, sem_b)
            c0.wait()
            do_adds(off0, buf_a, dref_a, sem_sa)
            c1.wait()
            do_adds(off1, buf_b, dref_b, sem_sb)
            drain(buf_a, sem_sa)
            drain(buf_b, sem_sb)
            return 0

        lax.fori_loop(0, PAIRS, gs_body, 0)
        plsc.subcore_barrier()
        pltpu.sync_copy(acc.at[pl.ds(s * WBR, WBR)],
                        out_h.at[pl.ds(g_lo + s * WBR, WBR)])


def _scatter_add(y, rowp, colp, zer):
    f = functools.partial(
        pl.kernel,
        mesh=_mesh,
        out_type=jax.ShapeDtypeStruct((NPAD, F), jnp.float32),
        scratch_types=[
            pltpu.VMEM((EPS,), jnp.int32),
            pltpu.VMEM((EPS,), jnp.int32),
            pltpu.VMEM((GWB, F), jnp.float32),
            pltpu.VMEM((GWB, F), jnp.float32),
            pltpu.VMEM((GWB,), jnp.int32),
            pltpu.VMEM((GWB,), jnp.int32),
            pltpu.VMEM_SHARED((ACC_R, F), jnp.float32),
            pltpu.SemaphoreType.DMA,
            pltpu.SemaphoreType.DMA,
            pltpu.SemaphoreType.DMA,
            pltpu.SemaphoreType.DMA,
        ],
    )(_scatter_body)
    return f(y, rowp, colp, zer)


def _scores_body(o_ref, b_ref, a1_ref, a2_ref, s1_ref, s2_ref):
    h = jax.nn.relu(o_ref[...] + b_ref[...])
    s1_ref[...] = jnp.sum(h * a1_ref[...], axis=1).reshape(8, F)
    s2_ref[...] = jnp.sum(h * a2_ref[...], axis=1).reshape(8, F)


def _scores(out0, bias2d, a1, a2):
    grid = (NB,)
    return pl.pallas_call(
        _scores_body,
        grid=grid,
        in_specs=[pl.BlockSpec((B_ROWS, F), lambda i: (i, 0)),
                  pl.BlockSpec((1, F), lambda i: (0, 0)),
                  pl.BlockSpec((1, F), lambda i: (0, 0)),
                  pl.BlockSpec((1, F), lambda i: (0, 0))],
        out_specs=[pl.BlockSpec((8, F), lambda i: (i, 0)),
                   pl.BlockSpec((8, F), lambda i: (i, 0))],
        out_shape=[jax.ShapeDtypeStruct((NPAD // F, F), jnp.float32),
                   jax.ShapeDtypeStruct((NPAD // F, F), jnp.float32)],
    )(out0, bias2d, a1, a2)


def _edge_body(row_h, col_h, s1_h, s2_h, e_h, row_v, col_v, a_v, b_v, sem):
    c = lax.axis_index("c")
    s = lax.axis_index("s")
    gw = c * 16 + s
    base = gw * EPW
    pltpu.sync_copy(row_h.at[pl.ds(base, EPW)], row_v)
    pltpu.sync_copy(col_h.at[pl.ds(base, EPW)], col_v)

    def gbody(w, _):
        off = w * GW
        c1 = pltpu.async_copy(s1_h.at[row_v.at[pl.ds(off, GW)]],
                              a_v.at[pl.ds(off, GW)], sem)
        c2 = pltpu.async_copy(s2_h.at[col_v.at[pl.ds(off, GW)]],
                              b_v.at[pl.ds(off, GW)], sem)
        c1.wait()
        c2.wait()
        return 0

    lax.fori_loop(0, NWIN, gbody, 0)

    def cbody(j, _):
        a = a_v[pl.ds(j * 16, 16)]
        b = b_v[pl.ds(j * 16, 16)]
        e = a + b
        a_v[pl.ds(j * 16, 16)] = jnp.where(e >= 0.0, e, e * 0.2)
        return 0

    lax.fori_loop(0, VREGS, cbody, 0)
    pltpu.sync_copy(a_v, e_h.at[pl.ds(base, EPW)])


def _edge_scores(rowp, colp, s1, s2):
    f = functools.partial(
        pl.kernel,
        mesh=_mesh,
        out_type=jax.ShapeDtypeStruct((EPAD,), jnp.float32),
        scratch_types=[
            pltpu.VMEM((EPW,), jnp.int32),
            pltpu.VMEM((EPW,), jnp.int32),
            pltpu.VMEM((EPW,), jnp.float32),
            pltpu.VMEM((EPW,), jnp.float32),
            pltpu.SemaphoreType.DMA,
        ],
    )(_edge_body)
    return f(rowp, colp, s1, s2)


def _softmax_body(e_ref, al_ref):
    r = lax.broadcasted_iota(jnp.int32, (EPAD // F, F), 0)
    l = lax.broadcasted_iota(jnp.int32, (EPAD // F, F), 1)
    valid = (r * F + l) < E
    em = jnp.where(valid, e_ref[...], -1e30)
    m = jnp.max(em)
    p = jnp.where(valid, jnp.exp(em - m), 0.0)
    al_ref[...] = p * (1.0 / jnp.sum(p))


def _softmax(e2d):
    return pl.pallas_call(
        _softmax_body,
        out_shape=jax.ShapeDtypeStruct((EPAD // F, F), jnp.float32),
    )(e2d)


def _scale_body(o_ref, b_ref, al_ref, out_ref):
    h = jax.nn.relu(o_ref[...] + b_ref[...])
    out_ref[...] = h * al_ref[...]


def _scale(out0, bias2d, alpha_col):
    grid = (pl.cdiv(N, B_ROWS),)
    return pl.pallas_call(
        _scale_body,
        grid=grid,
        in_specs=[pl.BlockSpec((B_ROWS, F), lambda i: (i, 0)),
                  pl.BlockSpec((1, F), lambda i: (0, 0)),
                  pl.BlockSpec((B_ROWS, 1), lambda i: (i, 0))],
        out_specs=pl.BlockSpec((B_ROWS, F), lambda i: (i, 0)),
        out_shape=jax.ShapeDtypeStruct((N, F), jnp.float32),
    )(out0, bias2d, alpha_col)


def kernel(x, edge_index, weight, att, bias):
    row32 = edge_index[0].astype(jnp.int32)
    col32 = edge_index[1].astype(jnp.int32)
    pad = EPAD - E
    pad_i = jnp.arange(pad, dtype=jnp.int32)
    rowp = jnp.concatenate([row32, N + pad_i % (NPAD - N)])
    colp = jnp.concatenate([col32, pad_i % N])
    zer = jnp.zeros((WBR, F), jnp.float32)
    bias2d = bias.reshape(1, F)
    a1 = att[:, :F]
    a2 = att[:, F:]

    y = _matmul(x, weight)
    out0 = _scatter_add(y, rowp, colp, zer)
    s1, s2 = _scores(out0, bias2d, a1, a2)
    e = _edge_scores(rowp, colp, s1.reshape(NPAD), s2.reshape(NPAD))
    alpha2d = _softmax(e.reshape(EPAD // F, F))
    out = _scale(out0, bias2d, alpha2d.reshape(EPAD, 1))
    return out


# speedup vs baseline: 1.1096x; 1.1034x over previous
import functools

import jax
import jax.numpy as jnp
from jax import lax
from jax.experimental import pallas as pl
from jax.experimental.pallas import tpu as pltpu
from jax.experimental.pallas import tpu_sc as plsc

N = 100000
E = 100000
F = 128

NPAD = 102400
EPAD = 102400
NG = 10
GR = NPAD // NG
ACC_R = GR + 16
WBR = GR // 16
EPS = EPAD // 16
GWB = 128
PAIRS = EPS // (2 * GWB)
EPW = EPAD // 32
GW = 128
NWIN = EPW // GW
VREGS = EPW // 16
B_ROWS = 2048
NB = NPAD // B_ROWS

_mesh = plsc.VectorSubcoreMesh(core_axis_name="c", subcore_axis_name="s",
                               num_cores=2)


def _mm_body(x_ref, w_ref, y_ref):
    y_ref[...] = jnp.dot(x_ref[...], w_ref[...],
                         preferred_element_type=jnp.float32)


def _matmul(x, w):
    grid = (pl.cdiv(N, B_ROWS),)
    return pl.pallas_call(
        _mm_body,
        grid=grid,
        in_specs=[pl.BlockSpec((B_ROWS, F), lambda i: (i, 0)),
                  pl.BlockSpec((F, F), lambda i: (0, 0))],
        out_specs=pl.BlockSpec((B_ROWS, F), lambda i: (i, 0)),
        out_shape=jax.ShapeDtypeStruct((N, F), jnp.float32),
    )(x, w)


def _scatter_body(y_h, row_h, col_h, zer_h, out_h,
                  row_v, col_v, buf_a, buf_b, dref_a, dref_b, acc,
                  sem_a, sem_b, sem_sa, sem_sb):
    c = lax.axis_index("c")
    s = lax.axis_index("s")
    base = s * EPS
    pltpu.sync_copy(row_h.at[pl.ds(base, EPS)], row_v)
    pltpu.sync_copy(col_h.at[pl.ds(base, EPS)], col_v)
    zer16 = jnp.full((16,), 0, jnp.int32)
    lane = lax.iota(jnp.int32, 16)
    dummy_v = jnp.full((16,), GR, jnp.int32) + lane

    for p in range(NG // 2):
        g = c * (NG // 2) + p
        g_lo = g * GR
        lo_v = zer16 + g_lo
        hi_v = lo_v + GR

        pltpu.sync_copy(zer_h, acc.at[pl.ds(s * WBR, WBR)])

        @pl.when(s == 0)
        def _zero_dummy():
            pltpu.sync_copy(zer_h.at[pl.ds(0, 16)],
                            acc.at[pl.ds(GR, 16)])

        plsc.subcore_barrier()

        def do_adds(off, buf, dref, sem_s):
            for jj in range(GWB // 16):
                r = row_v[pl.ds(off + jj * 16, 16)]
                m = (r >= lo_v) & (r < hi_v)
                dref[pl.ds(jj * 16, 16)] = jnp.where(m, r - lo_v, dummy_v)
            pltpu.async_copy(buf, acc.at[dref], sem_s, add=True)

        def drain(buf, sem_s):
            pltpu.make_async_copy(buf, acc.at[pl.ds(0, GWB)], sem_s).wait()

        def gs_body(q, _):
            off0 = q * (2 * GWB)
            off1 = off0 + GWB
            c0 = pltpu.async_copy(
                y_h.at[col_v.at[pl.ds(off0, GWB)]], buf_a, sem_a)
            c1 = pltpu.async_copy(
                y_h.at[col_v.at[pl.ds(off1, GWB)]], buf_b, sem_b)
            c0.wait()
            do_adds(off0, buf_a, dref_a, sem_sa)
            c1.wait()
            do_adds(off1, buf_b, dref_b, sem_sb)
            drain(buf_a, sem_sa)
            drain(buf_b, sem_sb)
            return 0

        lax.fori_loop(0, PAIRS, gs_body, 0)
        plsc.subcore_barrier()
        pltpu.sync_copy(acc.at[pl.ds(s * WBR, WBR)],
                        out_h.at[pl.ds(g_lo + s * WBR, WBR)])


def _scatter_add(y, rowp, colp, zer):
    f = functools.partial(
        pl.kernel,
        mesh=_mesh,
        out_type=jax.ShapeDtypeStruct((NPAD, F), jnp.float32),
        scratch_types=[
            pltpu.VMEM((EPS,), jnp.int32),
            pltpu.VMEM((EPS,), jnp.int32),
            pltpu.VMEM((GWB, F), jnp.float32),
            pltpu.VMEM((GWB, F), jnp.float32),
            pltpu.VMEM((GWB,), jnp.int32),
            pltpu.VMEM((GWB,), jnp.int32),
            pltpu.VMEM_SHARED((ACC_R, F), jnp.float32),
            pltpu.SemaphoreType.DMA,
            pltpu.SemaphoreType.DMA,
            pltpu.SemaphoreType.DMA,
            pltpu.SemaphoreType.DMA,
        ],
    )(_scatter_body)
    return f(y, rowp, colp, zer)


def _scores_body(o_ref, b_ref, a1_ref, a2_ref, s1_ref, s2_ref):
    h = jax.nn.relu(o_ref[...] + b_ref[...])
    s1_ref[...] = jnp.sum(h * a1_ref[...], axis=1).reshape(B_ROWS // F, F)
    s2_ref[...] = jnp.sum(h * a2_ref[...], axis=1).reshape(B_ROWS // F, F)


def _scores(out0, bias2d, a1, a2):
    grid = (NB,)
    return pl.pallas_call(
        _scores_body,
        grid=grid,
        in_specs=[pl.BlockSpec((B_ROWS, F), lambda i: (i, 0)),
                  pl.BlockSpec((1, F), lambda i: (0, 0)),
                  pl.BlockSpec((1, F), lambda i: (0, 0)),
                  pl.BlockSpec((1, F), lambda i: (0, 0))],
        out_specs=[pl.BlockSpec((B_ROWS // F, F), lambda i: (i, 0)),
                   pl.BlockSpec((B_ROWS // F, F), lambda i: (i, 0))],
        out_shape=[jax.ShapeDtypeStruct((NPAD // F, F), jnp.float32),
                   jax.ShapeDtypeStruct((NPAD // F, F), jnp.float32)],
    )(out0, bias2d, a1, a2)


def _edge_body(row_h, col_h, s1_h, s2_h, e_h, row_v, col_v, a_v, b_v, sem):
    c = lax.axis_index("c")
    s = lax.axis_index("s")
    gw = c * 16 + s
    base = gw * EPW
    pltpu.sync_copy(row_h.at[pl.ds(base, EPW)], row_v)
    pltpu.sync_copy(col_h.at[pl.ds(base, EPW)], col_v)

    def gbody(w, _):
        off = w * GW
        c1 = pltpu.async_copy(s1_h.at[row_v.at[pl.ds(off, GW)]],
                              a_v.at[pl.ds(off, GW)], sem)
        c2 = pltpu.async_copy(s2_h.at[col_v.at[pl.ds(off, GW)]],
                              b_v.at[pl.ds(off, GW)], sem)
        c1.wait()
        c2.wait()
        return 0

    lax.fori_loop(0, NWIN, gbody, 0)

    def cbody(j, _):
        a = a_v[pl.ds(j * 16, 16)]
        b = b_v[pl.ds(j * 16, 16)]
        e = a + b
        a_v[pl.ds(j * 16, 16)] = jnp.where(e >= 0.0, e, e * 0.2)
        return 0

    lax.fori_loop(0, VREGS, cbody, 0)
    pltpu.sync_copy(a_v, e_h.at[pl.ds(base, EPW)])


def _edge_scores(rowp, colp, s1, s2):
    f = functools.partial(
        pl.kernel,
        mesh=_mesh,
        out_type=jax.ShapeDtypeStruct((EPAD,), jnp.float32),
        scratch_types=[
            pltpu.VMEM((EPW,), jnp.int32),
            pltpu.VMEM((EPW,), jnp.int32),
            pltpu.VMEM((EPW,), jnp.float32),
            pltpu.VMEM((EPW,), jnp.float32),
            pltpu.SemaphoreType.DMA,
        ],
    )(_edge_body)
    return f(rowp, colp, s1, s2)


def _softmax_body(e_ref, al_ref):
    r = lax.broadcasted_iota(jnp.int32, (EPAD // F, F), 0)
    l = lax.broadcasted_iota(jnp.int32, (EPAD // F, F), 1)
    valid = (r * F + l) < E
    em = jnp.where(valid, e_ref[...], -1e30)
    m = jnp.max(em)
    p = jnp.where(valid, jnp.exp(em - m), 0.0)
    al_ref[...] = p * (1.0 / jnp.sum(p))


def _softmax(e2d):
    return pl.pallas_call(
        _softmax_body,
        out_shape=jax.ShapeDtypeStruct((EPAD // F, F), jnp.float32),
    )(e2d)


def _scale_body(o_ref, b_ref, al_ref, out_ref):
    h = jax.nn.relu(o_ref[...] + b_ref[...])
    out_ref[...] = h * al_ref[...]


def _scale(out0, bias2d, alpha_col):
    grid = (pl.cdiv(N, B_ROWS),)
    return pl.pallas_call(
        _scale_body,
        grid=grid,
        in_specs=[pl.BlockSpec((B_ROWS, F), lambda i: (i, 0)),
                  pl.BlockSpec((1, F), lambda i: (0, 0)),
                  pl.BlockSpec((B_ROWS, 1), lambda i: (i, 0))],
        out_specs=pl.BlockSpec((B_ROWS, F), lambda i: (i, 0)),
        out_shape=jax.ShapeDtypeStruct((N, F), jnp.float32),
    )(out0, bias2d, alpha_col)


def kernel(x, edge_index, weight, att, bias):
    row32 = edge_index[0].astype(jnp.int32)
    col32 = edge_index[1].astype(jnp.int32)
    pad = EPAD - E
    pad_i = jnp.arange(pad, dtype=jnp.int32)
    rowp = jnp.concatenate([row32, N + pad_i % (NPAD - N)])
    colp = jnp.concatenate([col32, pad_i % N])
    zer = jnp.zeros((WBR, F), jnp.float32)
    bias2d = bias.reshape(1, F)
    a1 = att[:, :F]
    a2 = att[:, F:]

    y = _matmul(x, weight)
    out0 = _scatter_add(y, rowp, colp, zer)
    s1, s2 = _scores(out0, bias2d, a1, a2)
    e = _edge_scores(rowp, colp, s1.reshape(NPAD), s2.reshape(NPAD))
    alpha2d = _softmax(e.reshape(EPAD // F, F))
    out = _scale(out0, bias2d, alpha2d.reshape(EPAD, 1))
    return out
